# baseline (speedup 1.0000x reference)
import functools

import jax
import jax.numpy as jnp
from jax import lax
from jax.experimental import pallas as pl
from jax.experimental.pallas import tpu as pltpu
from jax.experimental.pallas import tpu_sc as plsc

B = 128
V = 100000
C = 0.1
HALF = 50048
CHUNK = 6400
TAIL = 5120
SPAN = 7 * CHUNK + TAIL
NTILE = 32
STRIP_A = SPAN
STRIP_B = HALF + SPAN


def _relu_sum_rows(buf_v, corrc, acc, unroll, iters):
    for r in range(8):
        cc = corrc[r]

        def body(i, a, r=r, cc=cc):
            base = i * (16 * unroll)
            for u in range(unroll):
                x = buf_v[r, pl.ds(base + 16 * u, 16)]
                a = a + jnp.maximum(x - cc, 0.0)
            return a

        acc = lax.fori_loop(0, iters, body, acc)
    return acc


def _sc_loss_body(target_hbm, pred_hbm, out_hbm, corr_hbm,
                  tgt_v, win_v, buf0_v, buf1_v, part_v, cv_v,
                  sem0, sem1, gsem):
    s = lax.axis_index("s")
    c = lax.axis_index("c")
    wid = s * 2 + c
    row0 = pl.multiple_of(8 * s, 8)
    col0 = pl.multiple_of(c * HALF, 128)

    tstart = jnp.minimum(row0, B - 16)
    pltpu.sync_copy(target_hbm.at[pl.ds(tstart, 16)], tgt_v)
    ii = lax.iota(jnp.int32, 16)
    pos = jnp.minimum((row0 - tstart) + ii, 15)
    tvv = tgt_v[...].at[pos].get(mode="promise_in_bounds")
    handles = []
    offs = []
    for r in range(8):
        t = tvv[r]
        cs = (t // 16) * 16
        handles.append(
            pltpu.async_copy(pred_hbm.at[row0 + r, pl.ds(cs, 16)],
                             win_v.at[r], gsem)
        )
        offs.append(t - cs)
    for h in handles:
        h.wait()
    corrc = []
    for r in range(8):
        off = jnp.full((16,), offs[r], jnp.int32)
        g = win_v[r, :].at[off].get(mode="promise_in_bounds")
        corrc.append(g - C)

    @pl.when(c == 0)
    def _():
        cv = jnp.zeros((16,), jnp.float32)
        for r in range(8):
            cv = jnp.where(ii == r, corrc[r], cv)
        cv_v[...] = cv
        pltpu.sync_copy(cv_v.at[pl.ds(0, 8)], corr_hbm.at[pl.ds(row0, 8)])

    bufs = [buf0_v, buf1_v]
    sems = [sem0, sem1]
    widths = [CHUNK] * 7 + [TAIL]
    starts = [col0 + 7 * CHUNK, col0]
    starts = []
    o = 0
    for w in widths:
        starts.append(o)
        o += w

    def issue(k):
        w = widths[k]
        return pltpu.async_copy(
            pred_hbm.at[pl.ds(row0, 8),
                        pl.ds(pl.multiple_of(col0 + starts[k], 128), w)],
            bufs[k % 2].at[:, pl.ds(0, w)],
            sems[k % 2],
        )

    acc = jnp.zeros((16,), jnp.float32)
    h_cur = issue(0)
    for k in range(8):
        h_next = issue(k + 1) if k + 1 < 8 else None
        h_cur.wait()
        if widths[k] == CHUNK:
            acc = _relu_sum_rows(bufs[k % 2], corrc, acc, 8, 50)
        else:
            acc = _relu_sum_rows(bufs[k % 2], corrc, acc, 8, 40)
        h_cur = h_next

    part_v[...] = acc
    pltpu.sync_copy(part_v, out_hbm.at[pl.ds(wid * 16, 16)])


_sc_loss = functools.partial(
    pl.kernel,
    mesh=plsc.VectorSubcoreMesh(core_axis_name="c", subcore_axis_name="s"),
    out_type=(
        jax.ShapeDtypeStruct((NTILE * 16,), jnp.float32),
        jax.ShapeDtypeStruct((B,), jnp.float32),
    ),
    scratch_types=[
        pltpu.VMEM((16,), jnp.int32),
        pltpu.VMEM((8, 16), jnp.float32),
        pltpu.VMEM((8, CHUNK), jnp.float32),
        pltpu.VMEM((8, CHUNK), jnp.float32),
        pltpu.VMEM((16,), jnp.float32),
        pltpu.VMEM((16,), jnp.float32),
        pltpu.SemaphoreType.DMA,
        pltpu.SemaphoreType.DMA,
        pltpu.SemaphoreType.DMA,
    ],
)(_sc_loss_body)


def _final_body(part_ref, corr_ref, pa_ref, pb_ref, out_ref):
    corr = corr_ref[...]
    sa = jnp.sum(jnp.maximum(pa_ref[...] - corr, 0.0))
    lanes = lax.broadcasted_iota(jnp.int32, (B, 128), 1)
    tb = jnp.maximum(pb_ref[...] - corr, 0.0)
    sb = jnp.sum(jnp.where(lanes < V - STRIP_B, tb, 0.0))
    out_ref[0] = (jnp.sum(part_ref[...]) + sa + sb - B * C) / B


def kernel(target, prediction):
    target = target.astype(jnp.int32)
    partials, corrc = _sc_loss(target, prediction)
    out = pl.pallas_call(
        _final_body,
        grid=(1,),
        in_specs=[
            pl.BlockSpec((NTILE, 16), lambda k: (0, 0)),
            pl.BlockSpec((B, 1), lambda k: (0, 0)),
            pl.BlockSpec((B, 128), lambda k: (0, STRIP_A // 128)),
            pl.BlockSpec((B, 128), lambda k: (0, STRIP_B // 128)),
        ],
        out_specs=pl.BlockSpec(memory_space=pltpu.SMEM),
        out_shape=jax.ShapeDtypeStruct((1,), jnp.float32),
    )(partials.reshape(NTILE, 16), corrc.reshape(B, 1), prediction, prediction)
    return out


# speedup vs baseline: 1.1579x; 1.1579x over previous
import functools

import jax
import jax.numpy as jnp
from jax import lax
from jax.experimental import pallas as pl
from jax.experimental.pallas import tpu as pltpu
from jax.experimental.pallas import tpu_sc as plsc

B = 128
V = 100000
C = 0.1
RG = 8
K = B // RG
NWORK = B // 16


def _sc_gather_body(target_hbm, pred_hbm, out_hbm, tgt_v, vals_v, diag_v, sem):
    wid = lax.axis_index("s") * 2 + lax.axis_index("c")

    @pl.when(wid < NWORK)
    def _():
        base = wid * 16
        pltpu.sync_copy(target_hbm.at[pl.ds(base, 16)], tgt_v)
        tv = tgt_v[...]
        handles = []
        offs = []
        for i in range(16):
            t = tv[i]
            cs = (t // 16) * 16
            handles.append(
                pltpu.async_copy(
                    pred_hbm.at[base + i, pl.ds(cs, 16)], vals_v.at[i], sem
                )
            )
            offs.append(t - cs)
        for h in handles:
            h.wait()
        ii = lax.iota(jnp.int32, 16)
        d = jnp.full((16,), -C, jnp.float32)
        for i in range(16):
            off = jnp.full((16,), offs[i], jnp.int32)
            g = vals_v[i, :].at[off].get(mode="promise_in_bounds")
            d = jnp.where(ii == i, g - C, d)
        diag_v[...] = d
        pltpu.sync_copy(diag_v, out_hbm.at[pl.ds(base, 16)])


_sc_gather = functools.partial(
    pl.kernel,
    mesh=plsc.VectorSubcoreMesh(core_axis_name="c", subcore_axis_name="s"),
    out_type=jax.ShapeDtypeStruct((B,), jnp.float32),
    scratch_types=[
        pltpu.VMEM((16,), jnp.int32),
        pltpu.VMEM((16, 16), jnp.float32),
        pltpu.VMEM((16,), jnp.float32),
        pltpu.SemaphoreType.DMA,
    ],
)(_sc_gather_body)


def _tc_body(corrc_ref, pred_ref, out_ref, acc_ref):
    k = pl.program_id(0)
    s = jnp.sum(jnp.maximum(pred_ref[...] - corrc_ref[...], 0.0))

    @pl.when(k == 0)
    def _():
        acc_ref[0] = s

    @pl.when(k > 0)
    def _():
        acc_ref[0] += s

    @pl.when(k == K - 1)
    def _():
        out_ref[0] = (acc_ref[0] - B * C) / B


def kernel(target, prediction):
    target = target.astype(jnp.int32)
    corrc = _sc_gather(target, prediction)
    out = pl.pallas_call(
        _tc_body,
        grid=(K,),
        in_specs=[
            pl.BlockSpec((RG, 1), lambda k: (k, 0)),
            pl.BlockSpec((RG, V), lambda k: (k, 0)),
        ],
        out_specs=pl.BlockSpec(memory_space=pltpu.SMEM),
        out_shape=jax.ShapeDtypeStruct((1,), jnp.float32),
        scratch_shapes=[pltpu.SMEM((1,), jnp.float32)],
    )(corrc.reshape(B, 1), prediction)
    return out
